# baseline (speedup 1.0000x reference)
import functools

import jax
import jax.numpy as jnp
from jax import lax
from jax.experimental import pallas as pl
from jax.experimental.pallas import tpu as pltpu
from jax.experimental.pallas import tpu_sc as plsc

N = 10000
E = 320000
F = 128
H = 8
D = 16
NC = 2
NS = 16
NW = NC * NS
EPW = E // NW
C = 80
NCHUNK = EPW // C
NB = 5
NBLK = NCHUNK // NB
FW = F + 16
RPT = N // NS

R = 1000
G = N // R


def _pre_body(x_ref, w_ref, alm_ref, arm_ref, featel_ref, er_ref):
    f = jnp.dot(x_ref[...], w_ref[...], preferred_element_type=jnp.float32)
    el = jnp.dot(f, alm_ref[...], preferred_element_type=jnp.float32)
    featel_ref[...] = jnp.concatenate([f, el], axis=1)
    er_ref[...] = jnp.dot(f, arm_ref[...], preferred_element_type=jnp.float32)


def _tc_pre(x, w, alm, arm):
    return pl.pallas_call(
        _pre_body,
        grid=(G,),
        in_specs=[pl.BlockSpec((R, F), lambda i: (i, 0)),
                  pl.BlockSpec((F, F), lambda i: (0, 0)),
                  pl.BlockSpec((F, 16), lambda i: (0, 0)),
                  pl.BlockSpec((F, 16), lambda i: (0, 0))],
        out_specs=[pl.BlockSpec((R, FW), lambda i: (i, 0)),
                   pl.BlockSpec((R, 16), lambda i: (i, 0))],
        out_shape=[jax.ShapeDtypeStruct((N, FW), jnp.float32),
                   jax.ShapeDtypeStruct((N, 16), jnp.float32)],
    )(x, w, alm, arm)


def _normalize(a0, a1, d0, d1, hp, b):
    num = a0 + a1
    den = d0 + d1
    expand = (lax.broadcasted_iota(jnp.int32, (H, F), 1) // D ==
              lax.broadcasted_iota(jnp.int32, (H, F), 0)).astype(jnp.float32)
    dfull = jnp.dot(den, expand, preferred_element_type=jnp.float32)
    return num / jnp.maximum(dfull, 1e-9) + hp + b


def _mid_body(a0, a1, d0, d1, hp, b, w, alm, arm,
              h1_ref, featel_ref, er_ref):
    rst = _normalize(a0[...], a1[...], d0[...], d1[...], hp[...], b[...])
    rst = jnp.where(rst > 0, rst, 0.01 * rst)
    h1_ref[...] = rst
    f = jnp.dot(rst, w[...], preferred_element_type=jnp.float32)
    el = jnp.dot(f, alm[...], preferred_element_type=jnp.float32)
    featel_ref[...] = jnp.concatenate([f, el], axis=1)
    er_ref[...] = jnp.dot(f, arm[...], preferred_element_type=jnp.float32)


def _tc_mid(a0, a1, d0, d1, hp, b, w, alm, arm):
    big = pl.BlockSpec((R, F), lambda i: (i, 0))
    sml = pl.BlockSpec((R, 8), lambda i: (i, 0))
    return pl.pallas_call(
        _mid_body,
        grid=(G,),
        in_specs=[big, big, sml, sml, big,
                  pl.BlockSpec((1, F), lambda i: (0, 0)),
                  pl.BlockSpec((F, F), lambda i: (0, 0)),
                  pl.BlockSpec((F, 16), lambda i: (0, 0)),
                  pl.BlockSpec((F, 16), lambda i: (0, 0))],
        out_specs=[big,
                   pl.BlockSpec((R, FW), lambda i: (i, 0)),
                   pl.BlockSpec((R, 16), lambda i: (i, 0))],
        out_shape=[jax.ShapeDtypeStruct((N, F), jnp.float32),
                   jax.ShapeDtypeStruct((N, FW), jnp.float32),
                   jax.ShapeDtypeStruct((N, 16), jnp.float32)],
    )(a0, a1, d0, d1, hp, b, w, alm, arm)


def _fin_body(a0, a1, d0, d1, hp, b, out_ref):
    out_ref[...] = _normalize(a0[...], a1[...], d0[...], d1[...],
                              hp[...], b[...])


def _tc_fin(a0, a1, d0, d1, hp, b):
    big = pl.BlockSpec((R, F), lambda i: (i, 0))
    sml = pl.BlockSpec((R, 8), lambda i: (i, 0))
    return pl.pallas_call(
        _fin_body,
        grid=(G,),
        in_specs=[big, big, sml, sml, big,
                  pl.BlockSpec((1, F), lambda i: (0, 0))],
        out_specs=big,
        out_shape=jax.ShapeDtypeStruct((N, F), jnp.float32),
    )(a0, a1, d0, d1, hp, b)


def _edge_body(featel_h, er_h, sdx_h, z_h, acc_o,
               featel_v, er_v, sdx_v, acc_sh, gsem, ssem, ssem2, isem):
    c = lax.axis_index("c")
    s = lax.axis_index("s")
    r0 = s * RPT
    pltpu.sync_copy(z_h.at[pl.ds(r0, RPT)], acc_sh.at[pl.ds(r0, RPT)])
    wid = c * NS + s
    plsc.subcore_barrier()

    iota16 = lax.iota(jnp.int32, 16)

    def issue_gathers(k):
        blk = k // NB
        jb = k - blk * NB
        bp = lax.rem(blk, 2)
        p = lax.rem(k, 3)
        pltpu.async_copy(featel_h.at[sdx_v.at[bp, jb]],
                         featel_v.at[pl.ds(p * C, C)], gsem)
        pltpu.async_copy(er_h.at[sdx_v.at[bp, NB + jb]],
                         er_v.at[pl.ds(p * C, C)], gsem)

    pltpu.sync_copy(sdx_h.at[wid, 0], sdx_v.at[0])
    issue_gathers(0)
    pltpu.async_copy(sdx_h.at[wid, 1], sdx_v.at[1], isem)
    issue_gathers(1)

    def chunk(j, carry):
        blk = j // NB
        jb = j - blk * NB
        bp = lax.rem(blk, 2)
        p = lax.rem(j, 3)

        pltpu.make_async_copy(featel_h.at[sdx_v.at[bp, jb]],
                              featel_v.at[pl.ds(p * C, C)], gsem).wait()
        pltpu.make_async_copy(er_h.at[sdx_v.at[bp, NB + jb]],
                              er_v.at[pl.ds(p * C, C)], gsem).wait()

        off = p * C

        def group(g):
            ev = iota16 + (off + g * 16)
            hvs = [jnp.full((16,), 128 + h, jnp.int32) for h in range(H)]
            els = [plsc.load_gather(featel_v, [ev, hv]) for hv in hvs]
            ers = [plsc.load_gather(er_v, [ev, jnp.full((16,), h, jnp.int32)])
                   for h in range(H)]
            es = [a + b for a, b in zip(els, ers)]
            es = [jnp.where(e > 0.0, e, 0.2 * e) for e in es]
            exs = [jnp.exp(e) for e in es]
            for h in range(H):
                plsc.store_scatter(featel_v,
                                   [ev, jnp.full((16,), 136 + h, jnp.int32)],
                                   exs[h])
            for h in range(H):
                cvs = [jnp.full((16,), h * D + d, jnp.int32)
                       for d in range(D)]
                fs = [plsc.load_gather(featel_v, [ev, cv]) for cv in cvs]
                ps = [f * exs[h] for f in fs]
                for d in range(D):
                    plsc.store_scatter(featel_v, [ev, cvs[d]], ps[d])

        for g in range(C // 16):
            group(g)

        ssel = lax.rem(j, 2)
        pltpu.async_copy(featel_v.at[pl.ds(off, C)],
                         acc_sh.at[sdx_v.at[bp, NB + jb]],
                         ssem.at[ssel], add=True)

        @pl.when(j > 0)
        def _():
            jp = j - 1
            blkp = jp // NB
            pltpu.make_async_copy(
                featel_v.at[pl.ds(lax.rem(jp, 3) * C, C)],
                acc_sh.at[sdx_v.at[lax.rem(blkp, 2), NB + jp - blkp * NB]],
                ssem.at[lax.rem(jp, 2)]).wait()

        @pl.when(jnp.logical_and(jb == 0,
                                 jnp.logical_and(j > 0, blk + 1 < NBLK)))
        def _():
            pltpu.async_copy(sdx_h.at[wid, blk + 1],
                             sdx_v.at[lax.rem(blk + 1, 2)], isem)

        k = j + 2
        @pl.when(k < NCHUNK)
        def _():
            kb = k // NB

            @pl.when(jnp.logical_and(k - kb * NB == 0, kb > 0))
            def _():
                pltpu.make_async_copy(sdx_h.at[wid, kb],
                                      sdx_v.at[lax.rem(kb, 2)], isem).wait()

            issue_gathers(k)

        return carry

    lax.fori_loop(0, NCHUNK, chunk, 0)

    jp = NCHUNK - 1
    blkp = jp // NB
    pltpu.make_async_copy(
        featel_v.at[pl.ds(lax.rem(jp, 3) * C, C)],
        acc_sh.at[sdx_v.at[lax.rem(blkp, 2), NB + jp - blkp * NB]],
        ssem.at[lax.rem(jp, 2)]).wait()
    plsc.subcore_barrier()
    pltpu.sync_copy(acc_sh.at[pl.ds(r0, RPT)], acc_o.at[c, pl.ds(r0, RPT)])


def _edge(featel, er16, sdx, z):
    mesh = plsc.VectorSubcoreMesh(core_axis_name="c", subcore_axis_name="s",
                                  num_cores=NC, num_subcores=NS)
    run = pl.kernel(
        _edge_body,
        out_type=jax.ShapeDtypeStruct((NC, N, FW), jnp.float32),
        mesh=mesh,
        compiler_params=pltpu.CompilerParams(use_tc_tiling_on_sc=False,
                                             needs_layout_passes=False),
        scratch_types=[
            pltpu.VMEM((3 * C, FW), jnp.float32),
            pltpu.VMEM((3 * C, 16), jnp.float32),
            pltpu.VMEM((2, 2 * NB, C), jnp.int32),
            pltpu.VMEM_SHARED((N, FW), jnp.float32),
            pltpu.SemaphoreType.DMA,
            pltpu.SemaphoreType.DMA((2,)),
            pltpu.SemaphoreType.DMA,
            pltpu.SemaphoreType.DMA,
        ],
    )
    return run(featel, er16, sdx, z)


def kernel(n_feat, edge_index, W0, al0, ar0, b0, W1, al1, ar1, b1):
    src = edge_index[0].astype(jnp.int32).reshape(NW, NBLK, NB, C)
    dst = edge_index[1].astype(jnp.int32).reshape(NW, NBLK, NB, C)
    sdx = jnp.concatenate([src, dst], axis=2)
    eye = (jnp.arange(F)[:, None] // D ==
           jnp.arange(16)[None, :]).astype(jnp.float32)
    alm0 = al0.reshape(-1)[:, None] * eye
    arm0 = ar0.reshape(-1)[:, None] * eye
    alm1 = al1.reshape(-1)[:, None] * eye
    arm1 = ar1.reshape(-1)[:, None] * eye
    z = jnp.zeros((N, FW), jnp.float32)

    featel1, er1 = _tc_pre(n_feat, W0, alm0, arm0)
    acc1 = _edge(featel1, er1, sdx, z)
    h1, featel2, er2 = _tc_mid(acc1[0, :, :F], acc1[1, :, :F],
                               acc1[0, :, F + 8:], acc1[1, :, F + 8:],
                               n_feat, b0.reshape(1, F), W1, alm1, arm1)
    acc2 = _edge(featel2, er2, sdx, z)
    out = _tc_fin(acc2[0, :, :F], acc2[1, :, :F],
                  acc2[0, :, F + 8:], acc2[1, :, F + 8:],
                  h1, b1.reshape(1, F))
    return out


# speedup vs baseline: 1.0050x; 1.0050x over previous
import functools

import jax
import jax.numpy as jnp
from jax import lax
from jax.experimental import pallas as pl
from jax.experimental.pallas import tpu as pltpu
from jax.experimental.pallas import tpu_sc as plsc

N = 10000
E = 320000
F = 128
H = 8
D = 16
NC = 2
NS = 16
NW = NC * NS
EPW = E // NW
C = 80
NCHUNK = EPW // C
NB = 5
NBLK = NCHUNK // NB
FW = F + 16
RPT = N // NS

R = 1000
G = N // R


def _pre_body(x_ref, w_ref, alm_ref, arm_ref, featel_ref, er_ref):
    f = jnp.dot(x_ref[...], w_ref[...], preferred_element_type=jnp.float32)
    el = jnp.dot(f, alm_ref[...], preferred_element_type=jnp.float32)
    featel_ref[...] = jnp.concatenate([f, el], axis=1)
    er_ref[...] = jnp.dot(f, arm_ref[...], preferred_element_type=jnp.float32)


def _tc_pre(x, w, alm, arm):
    return pl.pallas_call(
        _pre_body,
        grid=(G,),
        in_specs=[pl.BlockSpec((R, F), lambda i: (i, 0)),
                  pl.BlockSpec((F, F), lambda i: (0, 0)),
                  pl.BlockSpec((F, 16), lambda i: (0, 0)),
                  pl.BlockSpec((F, 16), lambda i: (0, 0))],
        out_specs=[pl.BlockSpec((R, FW), lambda i: (i, 0)),
                   pl.BlockSpec((R, 16), lambda i: (i, 0))],
        out_shape=[jax.ShapeDtypeStruct((N, FW), jnp.float32),
                   jax.ShapeDtypeStruct((N, 16), jnp.float32)],
    )(x, w, alm, arm)


def _normalize(a0, a1, d0, d1, hp, b):
    num = a0 + a1
    den = d0 + d1
    expand = (lax.broadcasted_iota(jnp.int32, (H, F), 1) // D ==
              lax.broadcasted_iota(jnp.int32, (H, F), 0)).astype(jnp.float32)
    dfull = jnp.dot(den, expand, preferred_element_type=jnp.float32)
    return num / jnp.maximum(dfull, 1e-9) + hp + b


def _mid_body(a0, a1, d0, d1, hp, b, w, alm, arm,
              h1_ref, featel_ref, er_ref):
    rst = _normalize(a0[...], a1[...], d0[...], d1[...], hp[...], b[...])
    rst = jnp.where(rst > 0, rst, 0.01 * rst)
    h1_ref[...] = rst
    f = jnp.dot(rst, w[...], preferred_element_type=jnp.float32)
    el = jnp.dot(f, alm[...], preferred_element_type=jnp.float32)
    featel_ref[...] = jnp.concatenate([f, el], axis=1)
    er_ref[...] = jnp.dot(f, arm[...], preferred_element_type=jnp.float32)


def _tc_mid(a0, a1, d0, d1, hp, b, w, alm, arm):
    big = pl.BlockSpec((R, F), lambda i: (i, 0))
    sml = pl.BlockSpec((R, 8), lambda i: (i, 0))
    return pl.pallas_call(
        _mid_body,
        grid=(G,),
        in_specs=[big, big, sml, sml, big,
                  pl.BlockSpec((1, F), lambda i: (0, 0)),
                  pl.BlockSpec((F, F), lambda i: (0, 0)),
                  pl.BlockSpec((F, 16), lambda i: (0, 0)),
                  pl.BlockSpec((F, 16), lambda i: (0, 0))],
        out_specs=[big,
                   pl.BlockSpec((R, FW), lambda i: (i, 0)),
                   pl.BlockSpec((R, 16), lambda i: (i, 0))],
        out_shape=[jax.ShapeDtypeStruct((N, F), jnp.float32),
                   jax.ShapeDtypeStruct((N, FW), jnp.float32),
                   jax.ShapeDtypeStruct((N, 16), jnp.float32)],
    )(a0, a1, d0, d1, hp, b, w, alm, arm)


def _fin_body(a0, a1, d0, d1, hp, b, out_ref):
    out_ref[...] = _normalize(a0[...], a1[...], d0[...], d1[...],
                              hp[...], b[...])


def _tc_fin(a0, a1, d0, d1, hp, b):
    big = pl.BlockSpec((R, F), lambda i: (i, 0))
    sml = pl.BlockSpec((R, 8), lambda i: (i, 0))
    return pl.pallas_call(
        _fin_body,
        grid=(G,),
        in_specs=[big, big, sml, sml, big,
                  pl.BlockSpec((1, F), lambda i: (0, 0))],
        out_specs=big,
        out_shape=jax.ShapeDtypeStruct((N, F), jnp.float32),
    )(a0, a1, d0, d1, hp, b)


def _edge_body(featel_h, er_h, sdx_h, z_h, acc_o,
               featel_v, er_v, sdx_v, acc_sh, gsem, ssem, ssem2, isem):
    c = lax.axis_index("c")
    s = lax.axis_index("s")
    r0 = s * RPT
    pltpu.sync_copy(z_h.at[pl.ds(r0, RPT)], acc_sh.at[pl.ds(r0, RPT)])
    wid = c * NS + s
    plsc.subcore_barrier()

    iota16 = lax.iota(jnp.int32, 16)

    def issue_gathers(k):
        blk = k // NB
        jb = k - blk * NB
        bp = lax.rem(blk, 2)
        p = lax.rem(k, 3)
        pltpu.async_copy(featel_h.at[sdx_v.at[bp, jb]],
                         featel_v.at[pl.ds(p * C, C)], gsem)
        pltpu.async_copy(er_h.at[sdx_v.at[bp, NB + jb]],
                         er_v.at[pl.ds(p * C, C)], gsem)

    pltpu.sync_copy(sdx_h.at[wid, 0], sdx_v.at[0])
    issue_gathers(0)
    pltpu.async_copy(sdx_h.at[wid, 1], sdx_v.at[1], isem)
    issue_gathers(1)

    def chunk(j, carry):
        blk = j // NB
        jb = j - blk * NB
        bp = lax.rem(blk, 2)
        p = lax.rem(j, 3)

        pltpu.make_async_copy(featel_h.at[sdx_v.at[bp, jb]],
                              featel_v.at[pl.ds(p * C, C)], gsem).wait()
        pltpu.make_async_copy(er_h.at[sdx_v.at[bp, NB + jb]],
                              er_v.at[pl.ds(p * C, C)], gsem).wait()

        off = p * C

        def group(g, carry2):
            ev = iota16 + (off + g * 16)
            hvs = [jnp.full((16,), 128 + h, jnp.int32) for h in range(H)]
            els = [plsc.load_gather(featel_v, [ev, hv]) for hv in hvs]
            ers = [plsc.load_gather(er_v, [ev, jnp.full((16,), h, jnp.int32)])
                   for h in range(H)]
            es = [a + b for a, b in zip(els, ers)]
            es = [jnp.where(e > 0.0, e, 0.2 * e) for e in es]
            exs = [jnp.exp(e) for e in es]
            for h in range(H):
                plsc.store_scatter(featel_v,
                                   [ev, jnp.full((16,), 136 + h, jnp.int32)],
                                   exs[h])
            for h in range(H):
                cvs = [jnp.full((16,), h * D + d, jnp.int32)
                       for d in range(D)]
                fs = [plsc.load_gather(featel_v, [ev, cv]) for cv in cvs]
                ps = [f * exs[h] for f in fs]
                for d in range(D):
                    plsc.store_scatter(featel_v, [ev, cvs[d]], ps[d])
            return carry2

        lax.fori_loop(0, C // 16, group, 0)

        ssel = lax.rem(j, 2)
        pltpu.async_copy(featel_v.at[pl.ds(off, C)],
                         acc_sh.at[sdx_v.at[bp, NB + jb]],
                         ssem.at[ssel], add=True)

        @pl.when(j > 0)
        def _():
            jp = j - 1
            blkp = jp // NB
            pltpu.make_async_copy(
                featel_v.at[pl.ds(lax.rem(jp, 3) * C, C)],
                acc_sh.at[sdx_v.at[lax.rem(blkp, 2), NB + jp - blkp * NB]],
                ssem.at[lax.rem(jp, 2)]).wait()

        @pl.when(jnp.logical_and(jb == 0,
                                 jnp.logical_and(j > 0, blk + 1 < NBLK)))
        def _():
            pltpu.async_copy(sdx_h.at[wid, blk + 1],
                             sdx_v.at[lax.rem(blk + 1, 2)], isem)

        k = j + 2
        @pl.when(k < NCHUNK)
        def _():
            kb = k // NB

            @pl.when(jnp.logical_and(k - kb * NB == 0, kb > 0))
            def _():
                pltpu.make_async_copy(sdx_h.at[wid, kb],
                                      sdx_v.at[lax.rem(kb, 2)], isem).wait()

            issue_gathers(k)

        return carry

    lax.fori_loop(0, NCHUNK, chunk, 0)

    jp = NCHUNK - 1
    blkp = jp // NB
    pltpu.make_async_copy(
        featel_v.at[pl.ds(lax.rem(jp, 3) * C, C)],
        acc_sh.at[sdx_v.at[lax.rem(blkp, 2), NB + jp - blkp * NB]],
        ssem.at[lax.rem(jp, 2)]).wait()
    plsc.subcore_barrier()
    pltpu.sync_copy(acc_sh.at[pl.ds(r0, RPT)], acc_o.at[c, pl.ds(r0, RPT)])


def _edge(featel, er16, sdx, z):
    mesh = plsc.VectorSubcoreMesh(core_axis_name="c", subcore_axis_name="s",
                                  num_cores=NC, num_subcores=NS)
    run = pl.kernel(
        _edge_body,
        out_type=jax.ShapeDtypeStruct((NC, N, FW), jnp.float32),
        mesh=mesh,
        compiler_params=pltpu.CompilerParams(use_tc_tiling_on_sc=False,
                                             needs_layout_passes=False),
        scratch_types=[
            pltpu.VMEM((3 * C, FW), jnp.float32),
            pltpu.VMEM((3 * C, 16), jnp.float32),
            pltpu.VMEM((2, 2 * NB, C), jnp.int32),
            pltpu.VMEM_SHARED((N, FW), jnp.float32),
            pltpu.SemaphoreType.DMA,
            pltpu.SemaphoreType.DMA((2,)),
            pltpu.SemaphoreType.DMA,
            pltpu.SemaphoreType.DMA,
        ],
    )
    return run(featel, er16, sdx, z)


def kernel(n_feat, edge_index, W0, al0, ar0, b0, W1, al1, ar1, b1):
    src = edge_index[0].astype(jnp.int32).reshape(NW, NBLK, NB, C)
    dst = edge_index[1].astype(jnp.int32).reshape(NW, NBLK, NB, C)
    sdx = jnp.concatenate([src, dst], axis=2)
    eye = (jnp.arange(F)[:, None] // D ==
           jnp.arange(16)[None, :]).astype(jnp.float32)
    alm0 = al0.reshape(-1)[:, None] * eye
    arm0 = ar0.reshape(-1)[:, None] * eye
    alm1 = al1.reshape(-1)[:, None] * eye
    arm1 = ar1.reshape(-1)[:, None] * eye
    z = jnp.zeros((N, FW), jnp.float32)

    featel1, er1 = _tc_pre(n_feat, W0, alm0, arm0)
    acc1 = _edge(featel1, er1, sdx, z)
    h1, featel2, er2 = _tc_mid(acc1[0, :, :F], acc1[1, :, :F],
                               acc1[0, :, F + 8:], acc1[1, :, F + 8:],
                               n_feat, b0.reshape(1, F), W1, alm1, arm1)
    acc2 = _edge(featel2, er2, sdx, z)
    out = _tc_fin(acc2[0, :, :F], acc2[1, :, :F],
                  acc2[0, :, F + 8:], acc2[1, :, F + 8:],
                  h1, b1.reshape(1, F))
    return out


# speedup vs baseline: 1.0596x; 1.0544x over previous
import functools

import jax
import jax.numpy as jnp
from jax import lax
from jax.experimental import pallas as pl
from jax.experimental.pallas import tpu as pltpu
from jax.experimental.pallas import tpu_sc as plsc

N = 10000
E = 320000
F = 128
H = 8
D = 16
NC = 2
NS = 16
NW = NC * NS
EPW = E // NW
C = 80
NCHUNK = EPW // C
NB = 5
NBLK = NCHUNK // NB
FW = F + 16
RPT = N // NS

R = 1000
G = N // R


def _pre_body(x_ref, w_ref, alm_ref, arm_ref, featel_ref, er_ref):
    f = jnp.dot(x_ref[...], w_ref[...], preferred_element_type=jnp.float32)
    el = jnp.dot(f, alm_ref[...], preferred_element_type=jnp.float32)
    featel_ref[...] = jnp.concatenate([f, el], axis=1)
    er_ref[...] = jnp.dot(f, arm_ref[...], preferred_element_type=jnp.float32)


def _tc_pre(x, w, alm, arm):
    return pl.pallas_call(
        _pre_body,
        grid=(G,),
        in_specs=[pl.BlockSpec((R, F), lambda i: (i, 0)),
                  pl.BlockSpec((F, F), lambda i: (0, 0)),
                  pl.BlockSpec((F, 16), lambda i: (0, 0)),
                  pl.BlockSpec((F, 16), lambda i: (0, 0))],
        out_specs=[pl.BlockSpec((R, FW), lambda i: (i, 0)),
                   pl.BlockSpec((R, 16), lambda i: (i, 0))],
        out_shape=[jax.ShapeDtypeStruct((N, FW), jnp.float32),
                   jax.ShapeDtypeStruct((N, 16), jnp.float32)],
    )(x, w, alm, arm)


def _normalize(a0, a1, d0, d1, hp, b):
    num = a0 + a1
    den = d0 + d1
    expand = (lax.broadcasted_iota(jnp.int32, (H, F), 1) // D ==
              lax.broadcasted_iota(jnp.int32, (H, F), 0)).astype(jnp.float32)
    dfull = jnp.dot(den, expand, preferred_element_type=jnp.float32)
    return num / jnp.maximum(dfull, 1e-9) + hp + b


def _mid_body(acc, hp, b, w, alm, arm, h1_ref, featel_ref, er_ref):
    rst = _normalize(acc[0, :, :F], acc[1, :, :F],
                     acc[0, :, F + 8:], acc[1, :, F + 8:], hp[...], b[...])
    rst = jnp.where(rst > 0, rst, 0.01 * rst)
    h1_ref[...] = rst
    f = jnp.dot(rst, w[...], preferred_element_type=jnp.float32)
    el = jnp.dot(f, alm[...], preferred_element_type=jnp.float32)
    featel_ref[...] = jnp.concatenate([f, el], axis=1)
    er_ref[...] = jnp.dot(f, arm[...], preferred_element_type=jnp.float32)


def _tc_mid(acc, hp, b, w, alm, arm):
    big = pl.BlockSpec((R, F), lambda i: (i, 0))
    return pl.pallas_call(
        _mid_body,
        grid=(G,),
        in_specs=[pl.BlockSpec((NC, R, FW), lambda i: (0, i, 0)),
                  big,
                  pl.BlockSpec((1, F), lambda i: (0, 0)),
                  pl.BlockSpec((F, F), lambda i: (0, 0)),
                  pl.BlockSpec((F, 16), lambda i: (0, 0)),
                  pl.BlockSpec((F, 16), lambda i: (0, 0))],
        out_specs=[big,
                   pl.BlockSpec((R, FW), lambda i: (i, 0)),
                   pl.BlockSpec((R, 16), lambda i: (i, 0))],
        out_shape=[jax.ShapeDtypeStruct((N, F), jnp.float32),
                   jax.ShapeDtypeStruct((N, FW), jnp.float32),
                   jax.ShapeDtypeStruct((N, 16), jnp.float32)],
    )(acc, hp, b, w, alm, arm)


def _fin_body(acc, hp, b, out_ref):
    out_ref[...] = _normalize(acc[0, :, :F], acc[1, :, :F],
                              acc[0, :, F + 8:], acc[1, :, F + 8:],
                              hp[...], b[...])


def _tc_fin(acc, hp, b):
    big = pl.BlockSpec((R, F), lambda i: (i, 0))
    return pl.pallas_call(
        _fin_body,
        grid=(G,),
        in_specs=[pl.BlockSpec((NC, R, FW), lambda i: (0, i, 0)),
                  big,
                  pl.BlockSpec((1, F), lambda i: (0, 0))],
        out_specs=big,
        out_shape=jax.ShapeDtypeStruct((N, F), jnp.float32),
    )(acc, hp, b)


def _edge_body(featel_h, er_h, sdx_h, z_h, acc_o,
               featel_v, er_v, sdx_v, acc_sh, gsem, ssem, ssem2, isem):
    c = lax.axis_index("c")
    s = lax.axis_index("s")
    r0 = s * RPT
    pltpu.sync_copy(z_h.at[pl.ds(r0, RPT)], acc_sh.at[pl.ds(r0, RPT)])
    wid = c * NS + s
    plsc.subcore_barrier()

    iota16 = lax.iota(jnp.int32, 16)

    def issue_gathers(k):
        blk = k // NB
        jb = k - blk * NB
        bp = lax.rem(blk, 2)
        p = lax.rem(k, 3)
        pltpu.async_copy(featel_h.at[sdx_v.at[bp, jb]],
                         featel_v.at[pl.ds(p * C, C)], gsem)
        pltpu.async_copy(er_h.at[sdx_v.at[bp, NB + jb]],
                         er_v.at[pl.ds(p * C, C)], gsem)

    pltpu.sync_copy(sdx_h.at[wid, 0], sdx_v.at[0])
    issue_gathers(0)
    pltpu.async_copy(sdx_h.at[wid, 1], sdx_v.at[1], isem)
    issue_gathers(1)

    def chunk(j, carry):
        blk = j // NB
        jb = j - blk * NB
        bp = lax.rem(blk, 2)
        p = lax.rem(j, 3)

        pltpu.make_async_copy(featel_h.at[sdx_v.at[bp, jb]],
                              featel_v.at[pl.ds(p * C, C)], gsem).wait()
        pltpu.make_async_copy(er_h.at[sdx_v.at[bp, NB + jb]],
                              er_v.at[pl.ds(p * C, C)], gsem).wait()

        off = p * C

        def group(g, carry2):
            ev = iota16 + (off + g * 16)
            hvs = [jnp.full((16,), 128 + h, jnp.int32) for h in range(H)]
            els = [plsc.load_gather(featel_v, [ev, hv]) for hv in hvs]
            ers = [plsc.load_gather(er_v, [ev, jnp.full((16,), h, jnp.int32)])
                   for h in range(H)]
            es = [a + b for a, b in zip(els, ers)]
            es = [jnp.where(e > 0.0, e, 0.2 * e) for e in es]
            exs = [jnp.exp(e) for e in es]
            for h in range(H):
                plsc.store_scatter(featel_v,
                                   [ev, jnp.full((16,), 136 + h, jnp.int32)],
                                   exs[h])
            for h in range(H):
                cvs = [jnp.full((16,), h * D + d, jnp.int32)
                       for d in range(D)]
                fs = [plsc.load_gather(featel_v, [ev, cv]) for cv in cvs]
                ps = [f * exs[h] for f in fs]
                for d in range(D):
                    plsc.store_scatter(featel_v, [ev, cvs[d]], ps[d])
            return carry2

        lax.fori_loop(0, C // 16, group, 0)

        ssel = lax.rem(j, 2)
        pltpu.async_copy(featel_v.at[pl.ds(off, C)],
                         acc_sh.at[sdx_v.at[bp, NB + jb]],
                         ssem.at[ssel], add=True)

        @pl.when(j > 0)
        def _():
            jp = j - 1
            blkp = jp // NB
            pltpu.make_async_copy(
                featel_v.at[pl.ds(lax.rem(jp, 3) * C, C)],
                acc_sh.at[sdx_v.at[lax.rem(blkp, 2), NB + jp - blkp * NB]],
                ssem.at[lax.rem(jp, 2)]).wait()

        @pl.when(jnp.logical_and(jb == 0,
                                 jnp.logical_and(j > 0, blk + 1 < NBLK)))
        def _():
            pltpu.async_copy(sdx_h.at[wid, blk + 1],
                             sdx_v.at[lax.rem(blk + 1, 2)], isem)

        k = j + 2
        @pl.when(k < NCHUNK)
        def _():
            kb = k // NB

            @pl.when(jnp.logical_and(k - kb * NB == 0, kb > 0))
            def _():
                pltpu.make_async_copy(sdx_h.at[wid, kb],
                                      sdx_v.at[lax.rem(kb, 2)], isem).wait()

            issue_gathers(k)

        return carry

    lax.fori_loop(0, NCHUNK, chunk, 0)

    jp = NCHUNK - 1
    blkp = jp // NB
    pltpu.make_async_copy(
        featel_v.at[pl.ds(lax.rem(jp, 3) * C, C)],
        acc_sh.at[sdx_v.at[lax.rem(blkp, 2), NB + jp - blkp * NB]],
        ssem.at[lax.rem(jp, 2)]).wait()
    plsc.subcore_barrier()
    pltpu.sync_copy(acc_sh.at[pl.ds(r0, RPT)], acc_o.at[c, pl.ds(r0, RPT)])


def _edge(featel, er16, sdx, z):
    mesh = plsc.VectorSubcoreMesh(core_axis_name="c", subcore_axis_name="s",
                                  num_cores=NC, num_subcores=NS)
    run = pl.kernel(
        _edge_body,
        out_type=jax.ShapeDtypeStruct((NC, N, FW), jnp.float32),
        mesh=mesh,
        compiler_params=pltpu.CompilerParams(use_tc_tiling_on_sc=False,
                                             needs_layout_passes=False),
        scratch_types=[
            pltpu.VMEM((3 * C, FW), jnp.float32),
            pltpu.VMEM((3 * C, 16), jnp.float32),
            pltpu.VMEM((2, 2 * NB, C), jnp.int32),
            pltpu.VMEM_SHARED((N, FW), jnp.float32),
            pltpu.SemaphoreType.DMA,
            pltpu.SemaphoreType.DMA((2,)),
            pltpu.SemaphoreType.DMA,
            pltpu.SemaphoreType.DMA,
        ],
    )
    return run(featel, er16, sdx, z)


def kernel(n_feat, edge_index, W0, al0, ar0, b0, W1, al1, ar1, b1):
    src = edge_index[0].astype(jnp.int32).reshape(NW, NBLK, NB, C)
    dst = edge_index[1].astype(jnp.int32).reshape(NW, NBLK, NB, C)
    sdx = jnp.concatenate([src, dst], axis=2)
    eye = (jnp.arange(F)[:, None] // D ==
           jnp.arange(16)[None, :]).astype(jnp.float32)
    alm0 = al0.reshape(-1)[:, None] * eye
    arm0 = ar0.reshape(-1)[:, None] * eye
    alm1 = al1.reshape(-1)[:, None] * eye
    arm1 = ar1.reshape(-1)[:, None] * eye
    z = jnp.zeros((N, FW), jnp.float32)

    featel1, er1 = _tc_pre(n_feat, W0, alm0, arm0)
    acc1 = _edge(featel1, er1, sdx, z)
    h1, featel2, er2 = _tc_mid(acc1, n_feat, b0.reshape(1, F),
                               W1, alm1, arm1)
    acc2 = _edge(featel2, er2, sdx, z)
    out = _tc_fin(acc2, h1, b1.reshape(1, F))
    return out


# speedup vs baseline: 1.0597x; 1.0000x over previous
import jax
import jax.numpy as jnp
from jax import lax
from jax.experimental import pallas as pl
from jax.experimental.pallas import tpu as pltpu
from jax.experimental.pallas import tpu_sc as plsc

N = 10000
E = 320000
F = 128
H = 8
D = 16
NC = 2
NS = 16
NW = NC * NS
EPW = E // NW
C = 80
NCHUNK = EPW // C
NB = 5
NBLK = NCHUNK // NB
FW = F + 16
RPT = N // NS

R = 1000
G = N // R


def _pre_body(x_ref, w_ref, alm_ref, arm_ref, featel_ref, er_ref):
    f = jnp.dot(x_ref[...], w_ref[...], preferred_element_type=jnp.float32)
    el = jnp.dot(f, alm_ref[...], preferred_element_type=jnp.float32)
    featel_ref[...] = jnp.concatenate([f, el], axis=1)
    er_ref[...] = jnp.dot(f, arm_ref[...], preferred_element_type=jnp.float32)


def _tc_pre(x, w, alm, arm):
    return pl.pallas_call(
        _pre_body,
        grid=(G,),
        in_specs=[pl.BlockSpec((R, F), lambda i: (i, 0)),
                  pl.BlockSpec((F, F), lambda i: (0, 0)),
                  pl.BlockSpec((F, 16), lambda i: (0, 0)),
                  pl.BlockSpec((F, 16), lambda i: (0, 0))],
        out_specs=[pl.BlockSpec((R, FW), lambda i: (i, 0)),
                   pl.BlockSpec((R, 16), lambda i: (i, 0))],
        out_shape=[jax.ShapeDtypeStruct((N, FW), jnp.float32),
                   jax.ShapeDtypeStruct((N, 16), jnp.float32)],
    )(x, w, alm, arm)


def _normalize(a0, a1, d0, d1, hp, b):
    num = a0 + a1
    den = d0 + d1
    expand = (lax.broadcasted_iota(jnp.int32, (H, F), 1) // D ==
              lax.broadcasted_iota(jnp.int32, (H, F), 0)).astype(jnp.float32)
    dfull = jnp.dot(den, expand, preferred_element_type=jnp.float32)
    return num / jnp.maximum(dfull, 1e-9) + hp + b


def _mid_body(acc, hp, b, w, alm, arm, h1_ref, featel_ref, er_ref):
    rst = _normalize(acc[0, :, :F], acc[1, :, :F],
                     acc[0, :, F + 8:], acc[1, :, F + 8:], hp[...], b[...])
    rst = jnp.where(rst > 0, rst, 0.01 * rst)
    h1_ref[...] = rst
    f = jnp.dot(rst, w[...], preferred_element_type=jnp.float32)
    el = jnp.dot(f, alm[...], preferred_element_type=jnp.float32)
    featel_ref[...] = jnp.concatenate([f, el], axis=1)
    er_ref[...] = jnp.dot(f, arm[...], preferred_element_type=jnp.float32)


def _tc_mid(acc, hp, b, w, alm, arm):
    big = pl.BlockSpec((R, F), lambda i: (i, 0))
    return pl.pallas_call(
        _mid_body,
        grid=(G,),
        in_specs=[pl.BlockSpec((NC, R, FW), lambda i: (0, i, 0)),
                  big,
                  pl.BlockSpec((1, F), lambda i: (0, 0)),
                  pl.BlockSpec((F, F), lambda i: (0, 0)),
                  pl.BlockSpec((F, 16), lambda i: (0, 0)),
                  pl.BlockSpec((F, 16), lambda i: (0, 0))],
        out_specs=[big,
                   pl.BlockSpec((R, FW), lambda i: (i, 0)),
                   pl.BlockSpec((R, 16), lambda i: (i, 0))],
        out_shape=[jax.ShapeDtypeStruct((N, F), jnp.float32),
                   jax.ShapeDtypeStruct((N, FW), jnp.float32),
                   jax.ShapeDtypeStruct((N, 16), jnp.float32)],
    )(acc, hp, b, w, alm, arm)


def _fin_body(acc, hp, b, out_ref):
    out_ref[...] = _normalize(acc[0, :, :F], acc[1, :, :F],
                              acc[0, :, F + 8:], acc[1, :, F + 8:],
                              hp[...], b[...])


def _tc_fin(acc, hp, b):
    big = pl.BlockSpec((R, F), lambda i: (i, 0))
    return pl.pallas_call(
        _fin_body,
        grid=(G,),
        in_specs=[pl.BlockSpec((NC, R, FW), lambda i: (0, i, 0)),
                  big,
                  pl.BlockSpec((1, F), lambda i: (0, 0))],
        out_specs=big,
        out_shape=jax.ShapeDtypeStruct((N, F), jnp.float32),
    )(acc, hp, b)


def _edge_body(featel_h, er_h, sdx_h, z_h, acc_o,
               featel_v, er_v, sdx_v, acc_sh, gsem, ssem, ssem2, isem):
    c = lax.axis_index("c")
    s = lax.axis_index("s")
    r0 = s * RPT
    pltpu.sync_copy(z_h.at[pl.ds(r0, RPT)], acc_sh.at[pl.ds(r0, RPT)])
    wid = c * NS + s
    plsc.subcore_barrier()

    iota16 = lax.iota(jnp.int32, 16)

    def issue_gathers(k):
        blk = k // NB
        jb = k - blk * NB
        bp = lax.rem(blk, 2)
        p = lax.rem(k, 3)
        pltpu.async_copy(featel_h.at[sdx_v.at[bp, jb]],
                         featel_v.at[pl.ds(p * C, C)], gsem)
        pltpu.async_copy(er_h.at[sdx_v.at[bp, NB + jb]],
                         er_v.at[pl.ds(p * C, C)], gsem)

    pltpu.sync_copy(sdx_h.at[wid, 0], sdx_v.at[0])
    issue_gathers(0)
    pltpu.async_copy(sdx_h.at[wid, 1], sdx_v.at[1], isem)
    issue_gathers(1)

    def chunk(j, carry):
        blk = j // NB
        jb = j - blk * NB
        bp = lax.rem(blk, 2)
        p = lax.rem(j, 3)

        pltpu.make_async_copy(featel_h.at[sdx_v.at[bp, jb]],
                              featel_v.at[pl.ds(p * C, C)], gsem).wait()
        pltpu.make_async_copy(er_h.at[sdx_v.at[bp, NB + jb]],
                              er_v.at[pl.ds(p * C, C)], gsem).wait()

        off = p * C

        def group(g, carry2):
            ev = iota16 + (off + g * 16)
            hvs = [jnp.full((16,), 128 + h, jnp.int32) for h in range(H)]
            els = [plsc.load_gather(featel_v, [ev, hv]) for hv in hvs]
            ers = [plsc.load_gather(er_v, [ev, jnp.full((16,), h, jnp.int32)])
                   for h in range(H)]
            es = [a + b for a, b in zip(els, ers)]
            es = [jnp.where(e > 0.0, e, 0.2 * e) for e in es]
            exs = [jnp.exp(e) for e in es]
            for h in range(H):
                plsc.store_scatter(featel_v,
                                   [ev, jnp.full((16,), 136 + h, jnp.int32)],
                                   exs[h])
            for h in range(H):
                cvs = [jnp.full((16,), h * D + d, jnp.int32)
                       for d in range(D)]
                fs = [plsc.load_gather(featel_v, [ev, cv]) for cv in cvs]
                ps = [f * exs[h] for f in fs]
                for d in range(D):
                    plsc.store_scatter(featel_v, [ev, cvs[d]], ps[d])
            return carry2

        lax.fori_loop(0, C // 16, group, 0)

        ssel = lax.rem(j, 2)
        pltpu.async_copy(featel_v.at[pl.ds(off, C)],
                         acc_sh.at[sdx_v.at[bp, NB + jb]],
                         ssem.at[ssel], add=True)

        @pl.when(j > 0)
        def _():
            jp = j - 1
            blkp = jp // NB
            pltpu.make_async_copy(
                featel_v.at[pl.ds(lax.rem(jp, 3) * C, C)],
                acc_sh.at[sdx_v.at[lax.rem(blkp, 2), NB + jp - blkp * NB]],
                ssem.at[lax.rem(jp, 2)]).wait()

        @pl.when(jnp.logical_and(jb == 0,
                                 jnp.logical_and(j > 0, blk + 1 < NBLK)))
        def _():
            pltpu.async_copy(sdx_h.at[wid, blk + 1],
                             sdx_v.at[lax.rem(blk + 1, 2)], isem)

        k = j + 2
        @pl.when(k < NCHUNK)
        def _():
            kb = k // NB

            @pl.when(jnp.logical_and(k - kb * NB == 0, kb > 0))
            def _():
                pltpu.make_async_copy(sdx_h.at[wid, kb],
                                      sdx_v.at[lax.rem(kb, 2)], isem).wait()

            issue_gathers(k)

        return carry

    lax.fori_loop(0, NCHUNK, chunk, 0)

    jp = NCHUNK - 1
    blkp = jp // NB
    pltpu.make_async_copy(
        featel_v.at[pl.ds(lax.rem(jp, 3) * C, C)],
        acc_sh.at[sdx_v.at[lax.rem(blkp, 2), NB + jp - blkp * NB]],
        ssem.at[lax.rem(jp, 2)]).wait()
    plsc.subcore_barrier()
    pltpu.sync_copy(acc_sh.at[pl.ds(r0, RPT)], acc_o.at[c, pl.ds(r0, RPT)])


def _edge(featel, er16, sdx, z):
    mesh = plsc.VectorSubcoreMesh(core_axis_name="c", subcore_axis_name="s",
                                  num_cores=NC, num_subcores=NS)
    run = pl.kernel(
        _edge_body,
        out_type=jax.ShapeDtypeStruct((NC, N, FW), jnp.float32),
        mesh=mesh,
        compiler_params=pltpu.CompilerParams(use_tc_tiling_on_sc=False,
                                             needs_layout_passes=False),
        scratch_types=[
            pltpu.VMEM((3 * C, FW), jnp.float32),
            pltpu.VMEM((3 * C, 16), jnp.float32),
            pltpu.VMEM((2, 2 * NB, C), jnp.int32),
            pltpu.VMEM_SHARED((N, FW), jnp.float32),
            pltpu.SemaphoreType.DMA,
            pltpu.SemaphoreType.DMA((2,)),
            pltpu.SemaphoreType.DMA,
            pltpu.SemaphoreType.DMA,
        ],
    )
    return run(featel, er16, sdx, z)


def kernel(n_feat, edge_index, W0, al0, ar0, b0, W1, al1, ar1, b1):
    src = edge_index[0].astype(jnp.int32).reshape(NW, NBLK, NB, C)
    dst = edge_index[1].astype(jnp.int32).reshape(NW, NBLK, NB, C)
    sdx = jnp.concatenate([src, dst], axis=2)
    eye = (jnp.arange(F)[:, None] // D ==
           jnp.arange(16)[None, :]).astype(jnp.float32)
    alm0 = al0.reshape(-1)[:, None] * eye
    arm0 = ar0.reshape(-1)[:, None] * eye
    alm1 = al1.reshape(-1)[:, None] * eye
    arm1 = ar1.reshape(-1)[:, None] * eye
    z = jnp.zeros((N, FW), jnp.float32)

    featel1, er1 = _tc_pre(n_feat, W0, alm0, arm0)
    acc1 = _edge(featel1, er1, sdx, z)
    h1, featel2, er2 = _tc_mid(acc1, n_feat, b0.reshape(1, F),
                               W1, alm1, arm1)
    acc2 = _edge(featel2, er2, sdx, z)
    out = _tc_fin(acc2, h1, b1.reshape(1, F))
    return out
